# baseline (speedup 1.0000x reference)
import jax
import jax.numpy as jnp
from jax import lax
from jax.experimental import pallas as pl
from jax.experimental.pallas import tpu as pltpu
from jax.experimental.pallas import tpu_sc as plsc

D = 128
NEW_BASE = 100000
NUM_NEW = 1024

NC = 2
NS = 16
NW = NC * NS

B = 4096 * 200
PER_W = B // NW
CHUNK = 128
NCHUNK = PER_W // CHUNK
NBUF = 4
NVREG = CHUNK // 16


def _gather_body(x_hbm, emb_hbm, new_hbm, out_hbm,
                 idx_v, idx_c, rows_v, plist, prow, shared_new, *sems):
    gsems = sems[:NBUF]
    wsems = sems[NBUF:2 * NBUF]
    psem, ssem = sems[2 * NBUF:]
    sid = lax.axis_index("s")
    wid = sid * NC + lax.axis_index("c")
    row0 = wid * NCHUNK
    lanes = lax.iota(jnp.int32, 16)

    rows_per_tile = NUM_NEW // NS
    pltpu.sync_copy(new_hbm.at[pl.ds(sid * rows_per_tile, rows_per_tile)],
                    shared_new.at[pl.ds(sid * rows_per_tile, rows_per_tile)])
    plsc.subcore_barrier()

    pltpu.sync_copy(x_hbm.at[pl.ds(row0, NCHUNK)], idx_v)

    def out_slice(j):
        return out_hbm.at[pl.ds((row0 + j) * CHUNK, CHUNK)]

    def clamp_scan_chunk(j, b):
        for v in range(NVREG):
            ivec = idx_v[j, pl.ds(16 * v, 16)]
            idx_c[b, pl.ds(16 * v, 16)] = jnp.minimum(ivec, NEW_BASE - 1)
            plist[j, pl.ds(16 * v, 16)] = jnp.where(
                ivec >= NEW_BASE, ivec - NEW_BASE, -1)

    def patch_chunk(j):
        for v in range(NVREG):
            nraw = plist[j, pl.ds(16 * v, 16)]
            m = nraw >= 0
            npop = plsc.all_reduce_population_count(m)

            @pl.when(npop[0] > 0)
            def _():
                praw = (row0 + j) * CHUNK + 16 * v + lanes
                packed = jnp.bitwise_or(lax.shift_left(praw, 10), nraw)
                pk_max = jnp.max(jnp.where(m, packed, -1))
                nf = jnp.bitwise_and(pk_max, 1023)
                pf = lax.shift_right_logical(pk_max, 10)
                nvec = jnp.where(m, nraw, nf)
                pvec = jnp.where(m, praw, pf)
                pltpu.async_copy(shared_new.at[nvec], prow, psem)
                pltpu.make_async_copy(shared_new.at[nvec], prow, psem).wait()
                pltpu.async_copy(prow, out_hbm.at[pvec], ssem)
                pltpu.make_async_copy(prow, out_hbm.at[pvec], ssem).wait()

    for b in range(NBUF):
        clamp_scan_chunk(b, b)
        pltpu.async_copy(emb_hbm.at[idx_c.at[b]], rows_v.at[b], gsems[b])

    def step(i, carry):
        g0 = i * NBUF
        for b in range(NBUF):
            j = g0 + b
            pltpu.make_async_copy(
                emb_hbm.at[idx_c.at[b]], rows_v.at[b], gsems[b]
            ).wait()
            pltpu.async_copy(rows_v.at[b], out_slice(j), wsems[b])
            clamp_scan_chunk(j + NBUF, b)
            pltpu.make_async_copy(rows_v.at[b], out_slice(j), wsems[b]).wait()
            pltpu.async_copy(emb_hbm.at[idx_c.at[b]], rows_v.at[b], gsems[b])
            patch_chunk(j)
        return carry

    n_steady = NCHUNK // NBUF - 1
    lax.fori_loop(0, n_steady, step, 0)

    for b in range(NBUF):
        j = NCHUNK - NBUF + b
        pltpu.make_async_copy(
            emb_hbm.at[idx_c.at[b]], rows_v.at[b], gsems[b]
        ).wait()
        pltpu.async_copy(rows_v.at[b], out_slice(j), wsems[b])
    for b in range(NBUF):
        j = NCHUNK - NBUF + b
        pltpu.make_async_copy(rows_v.at[b], out_slice(j), wsems[b]).wait()
        patch_chunk(j)


_gather = pl.kernel(
    _gather_body,
    out_type=jax.ShapeDtypeStruct((B, D), jnp.float32),
    mesh=plsc.VectorSubcoreMesh(core_axis_name="c", subcore_axis_name="s"),
    compiler_params=pltpu.CompilerParams(needs_layout_passes=False),
    scratch_types=(
        [
            pltpu.VMEM((NCHUNK, CHUNK), jnp.int32),
            pltpu.VMEM((NBUF, CHUNK), jnp.int32),
            pltpu.VMEM((NBUF, CHUNK, D), jnp.float32),
            pltpu.VMEM((NCHUNK, CHUNK), jnp.int32),
            pltpu.VMEM((16, D), jnp.float32),
            pltpu.VMEM_SHARED((NUM_NEW, D), jnp.float32),
        ]
        + [pltpu.SemaphoreType.DMA] * (2 * NBUF + 2)
    ),
)


def kernel(x, embedding, new_embedding):
    x2d = x.reshape(B // CHUNK, CHUNK).astype(jnp.int32)
    out = _gather(x2d, embedding, new_embedding)
    return out.reshape(x.shape[0], x.shape[1], D)


# speedup vs baseline: 2.8248x; 2.8248x over previous
import jax
import jax.numpy as jnp
from jax import lax
from jax.experimental import pallas as pl
from jax.experimental.pallas import tpu as pltpu
from jax.experimental.pallas import tpu_sc as plsc

D = 128

NC = 2
NS = 16
NW = NC * NS

B = 4096 * 200
PER_W = B // NW
CHUNK = 128
NCHUNK = PER_W // CHUNK
NBUF = 4


def _gather_body(x_hbm, table_hbm, out_hbm, idx_v, rows_v, *sems):
    gsems = sems[:NBUF]
    wsems = sems[NBUF:]
    wid = lax.axis_index("s") * NC + lax.axis_index("c")
    row0 = wid * NCHUNK

    pltpu.sync_copy(x_hbm.at[pl.ds(row0, NCHUNK)], idx_v)

    def out_slice(j):
        return out_hbm.at[pl.ds((row0 + j) * CHUNK, CHUNK)]

    for b in range(NBUF):
        pltpu.async_copy(table_hbm.at[idx_v.at[b]], rows_v.at[b], gsems[b])

    def step(i, carry):
        g0 = i * NBUF
        for b in range(NBUF):
            j = g0 + b
            pltpu.make_async_copy(
                table_hbm.at[idx_v.at[j]], rows_v.at[b], gsems[b]
            ).wait()
            pltpu.async_copy(rows_v.at[b], out_slice(j), wsems[b])
            pltpu.make_async_copy(rows_v.at[b], out_slice(j), wsems[b]).wait()
            pltpu.async_copy(
                table_hbm.at[idx_v.at[j + NBUF]], rows_v.at[b], gsems[b]
            )
        return carry

    n_steady = NCHUNK // NBUF - 1
    lax.fori_loop(0, n_steady, step, 0)

    for b in range(NBUF):
        j = NCHUNK - NBUF + b
        pltpu.make_async_copy(
            table_hbm.at[idx_v.at[j]], rows_v.at[b], gsems[b]
        ).wait()
        pltpu.async_copy(rows_v.at[b], out_slice(j), wsems[b])
    for b in range(NBUF):
        j = NCHUNK - NBUF + b
        pltpu.make_async_copy(rows_v.at[b], out_slice(j), wsems[b]).wait()


_gather = pl.kernel(
    _gather_body,
    out_type=jax.ShapeDtypeStruct((B, D), jnp.float32),
    mesh=plsc.VectorSubcoreMesh(core_axis_name="c", subcore_axis_name="s"),
    compiler_params=pltpu.CompilerParams(needs_layout_passes=False),
    scratch_types=(
        [
            pltpu.VMEM((NCHUNK, CHUNK), jnp.int32),
            pltpu.VMEM((NBUF, CHUNK, D), jnp.float32),
        ]
        + [pltpu.SemaphoreType.DMA] * (2 * NBUF)
    ),
)


def kernel(x, embedding, new_embedding):
    table = jnp.concatenate([embedding, new_embedding], axis=0)
    x2d = x.reshape(B // CHUNK, CHUNK).astype(jnp.int32)
    out = _gather(x2d, table)
    return out.reshape(x.shape[0], x.shape[1], D)
